# baseline (speedup 1.0000x reference)
import functools

import jax
import jax.numpy as jnp
from jax import lax
from jax.experimental import pallas as pl
from jax.experimental.pallas import tpu as pltpu
from jax.experimental.pallas import tpu_sc as plsc

CUTOFF = 10.0
D = 128
NC, NS, L = 2, 16, 16
NW = NC * NS
CHUNK = 48
SUP = 8
NN = 10000
AGG_PAD = 10112
ROW_BLK = 400
EF_BLK = 4096


def _envelope_body(d_ref, c_ref):
    d = d_ref[...]
    c = 0.5 * (jnp.cos(d * (jnp.pi / CUTOFF)) + 1.0)
    c_ref[...] = c * (d < CUTOFF).astype(jnp.float32)


def _epre_body(ef_ref, w_ref, b_ref, o_ref):
    o_ref[...] = (
        jnp.dot(ef_ref[...], w_ref[...], preferred_element_type=jnp.float32)
        + b_ref[...]
    )


def _table_body(x_ref, w_ref, o_ref):
    o_ref[...] = jnp.dot(x_ref[...], w_ref[0],
                         preferred_element_type=jnp.float32)


def _update_body(x_ref, a0_ref, a1_ref, wx_ref, wg_ref, b_ref, o_ref):
    h = (
        jnp.dot(x_ref[...], wx_ref[...], preferred_element_type=jnp.float32)
        + jnp.dot(a0_ref[...] + a1_ref[...], wg_ref[...],
                  preferred_element_type=jnp.float32)
        + b_ref[...]
    )
    o_ref[...] = jnp.maximum(h, 0.0)


def _splat(v16, i):
    return lax.gather(
        v16, jnp.full((L, 1), i, jnp.int32),
        lax.GatherDimensionNumbers(offset_dims=(), collapsed_slice_dims=(0,),
                                   start_index_map=(0,)),
        slice_sizes=(1,), mode=lax.GatherScatterMode.PROMISE_IN_BOUNDS)


def _edge_body(t_hbm, epre_hbm, gidx_hbm, didx_hbm, c_hbm, zero_hbm,
               out_hbm, gbuf, ebuf, gidx, didx, cbuf, agg_sh,
               sem_g, sem_s, sem_i, *, n_sup):
    cid = lax.axis_index("c")
    sid = lax.axis_index("s")
    wid = cid * NS + sid
    nps = AGG_PAD // NS

    row0 = sid * nps
    pltpu.sync_copy(zero_hbm.at[pl.ds(row0, nps)], agg_sh.at[pl.ds(row0, nps)])
    plsc.subcore_barrier()

    chunk0 = wid * (n_sup * SUP)

    def fetch_idx(s, sp):
        r0 = chunk0 + s * SUP
        a = pltpu.async_copy(gidx_hbm.at[pl.ds(r0, SUP)], gidx.at[sp], sem_i)
        b = pltpu.async_copy(didx_hbm.at[pl.ds(r0, SUP)], didx.at[sp], sem_i)
        c = pltpu.async_copy(c_hbm.at[pl.ds(r0, SUP)], cbuf.at[sp], sem_i)
        return a, b, c

    def wait_idx(sp):
        pltpu.make_async_copy(gidx_hbm.at[pl.ds(0, SUP)], gidx.at[sp],
                              sem_i).wait()
        pltpu.make_async_copy(didx_hbm.at[pl.ds(0, SUP)], didx.at[sp],
                              sem_i).wait()
        pltpu.make_async_copy(c_hbm.at[pl.ds(0, SUP)], cbuf.at[sp],
                              sem_i).wait()

    def issue_ge(s, sp, j, p):
        r = chunk0 + s * SUP + j
        pltpu.async_copy(t_hbm.at[gidx.at[sp, j]], gbuf.at[p], sem_g)
        pltpu.async_copy(epre_hbm.at[pl.ds(r * CHUNK, CHUNK)], ebuf.at[p],
                         sem_g)

    def wait_ge(p):
        pltpu.make_async_copy(t_hbm.at[gidx.at[0, 0]], gbuf.at[p],
                              sem_g).wait()
        pltpu.make_async_copy(epre_hbm.at[pl.ds(0, CHUNK)], ebuf.at[p],
                              sem_g).wait()

    def issue_scatter(sp, j, p):
        pltpu.async_copy(ebuf.at[p], agg_sh.at[didx.at[sp, j]], sem_s,
                         add=True)

    def wait_scatter(sp, j, p):
        pltpu.make_async_copy(ebuf.at[p], agg_sh.at[didx.at[sp, j]],
                              sem_s).wait()

    def compute(sp, j, p):
        for g in range(CHUNK // L):
            cv16 = cbuf[sp, j, pl.ds(g * L, L)]

            def edge_fn(i, _):
                e = g * L + i
                cv = _splat(cv16, i)
                for f in range(D // L):
                    sl = pl.ds(f * L, L)
                    v = gbuf[p, 2 * e, sl] + gbuf[p, 2 * e + 1, sl] \
                        + ebuf[p, e, sl]
                    ebuf[p, e, sl] = jnp.maximum(v, 0.0) * cv
                return 0

            lax.fori_loop(0, L, edge_fn, 0)

    a, b, c = fetch_idx(0, 0)
    a.wait(); b.wait(); c.wait()
    issue_ge(0, 0, 0, 0)

    def sup_body(s, _):
        sp = lax.rem(s, 2)
        spn = 1 - sp
        for j in range(SUP):
            p = j % 2
            wait_ge(p)
            if j == 0:
                @pl.when(s > 0)
                def _():
                    wait_scatter(sp, SUP - 2, 1 - p)
                @pl.when(s + 1 < n_sup)
                def _():
                    fetch_idx(s + 1, spn)
            else:
                wait_scatter(sp, j - 1, 1 - p)
            if j < SUP - 1:
                issue_ge(s, sp, j + 1, 1 - p)
            elif j == SUP - 1:
                @pl.when(s + 1 < n_sup)
                def _():
                    wait_idx(spn)
                    issue_ge(s + 1, spn, 0, 1 - p)
            compute(sp, j, p)
            issue_scatter(sp, j, p)
        return 0

    lax.fori_loop(0, n_sup, sup_body, 0)
    wait_scatter(lax.rem(n_sup - 1, 2), SUP - 1, 1)
    plsc.subcore_barrier()
    pltpu.sync_copy(agg_sh.at[pl.ds(row0, nps)],
                    out_hbm.at[cid, pl.ds(row0, nps)])


def _make_edge_kernel(n_sup):
    mesh = plsc.VectorSubcoreMesh(core_axis_name="c", subcore_axis_name="s",
                                  num_cores=NC, num_subcores=NS)
    return pl.kernel(
        functools.partial(_edge_body, n_sup=n_sup),
        out_type=jax.ShapeDtypeStruct((NC, AGG_PAD, D), jnp.float32),
        mesh=mesh,
        scratch_types=[
            pltpu.VMEM((2, 2 * CHUNK, D), jnp.float32),
            pltpu.VMEM((2, CHUNK, D), jnp.float32),
            pltpu.VMEM((2, SUP, 2 * CHUNK), jnp.int32),
            pltpu.VMEM((2, SUP, CHUNK), jnp.int32),
            pltpu.VMEM((2, SUP, CHUNK), jnp.float32),
            pltpu.VMEM_SHARED((AGG_PAD, D), jnp.float32),
            pltpu.SemaphoreType.DMA,
            pltpu.SemaphoreType.DMA,
            pltpu.SemaphoreType.DMA,
        ],
    )


def kernel(node_feats, edge_feats, edge_index, dist, Wm, bm, Wa, ba):
    n, d_in = node_feats.shape
    e, d_edge = edge_feats.shape
    f32 = jnp.float32

    n_sup = -(-e // (NW * SUP * CHUNK))
    e_pad = NW * SUP * CHUNK * n_sup
    n_chunk_rows = e_pad // CHUNK

    x = node_feats
    ef = jnp.pad(edge_feats, ((0, e_pad - e), (0, 0)))
    src = jnp.pad(edge_index[0], (0, e_pad - e))
    dst = jnp.pad(edge_index[1], (0, e_pad - e))
    distp = jnp.pad(dist, (0, e_pad - e), constant_values=2.0 * CUTOFF)
    zeros = jnp.zeros((AGG_PAD, D), f32)

    gidx = jnp.stack([src, NN + dst], axis=-1).reshape(n_chunk_rows, 2 * CHUNK)
    didx = dst.reshape(n_chunk_rows, CHUNK)

    env = pl.pallas_call(
        _envelope_body,
        out_shape=jax.ShapeDtypeStruct((e_pad // D, D), f32),
    )(distp.reshape(e_pad // D, D))
    env = env.reshape(n_chunk_rows, CHUNK)

    n_row_blocks = NN // ROW_BLK
    table_call = pl.pallas_call(
        _table_body,
        grid=(2, n_row_blocks),
        in_specs=[
            pl.BlockSpec((ROW_BLK, D), lambda c, i: (i, 0)),
            pl.BlockSpec((1, D, D), lambda c, i: (c, 0, 0)),
        ],
        out_specs=pl.BlockSpec((ROW_BLK, D),
                               lambda c, i: (c * (NN // ROW_BLK) + i, 0)),
        out_shape=jax.ShapeDtypeStruct((2 * NN, D), f32),
    )

    epre_call = pl.pallas_call(
        _epre_body,
        grid=(e_pad // EF_BLK,),
        in_specs=[
            pl.BlockSpec((EF_BLK, d_edge), lambda i: (i, 0)),
            pl.BlockSpec((d_edge, D), lambda i: (0, 0)),
            pl.BlockSpec((1, D), lambda i: (0, 0)),
        ],
        out_specs=pl.BlockSpec((EF_BLK, D), lambda i: (i, 0)),
        out_shape=jax.ShapeDtypeStruct((e_pad, D), f32),
    )

    update_call = pl.pallas_call(
        _update_body,
        grid=(n_row_blocks,),
        in_specs=[
            pl.BlockSpec((ROW_BLK, D), lambda i: (i, 0)),
            pl.BlockSpec((ROW_BLK, D), lambda i: (i, 0)),
            pl.BlockSpec((ROW_BLK, D), lambda i: (i, 0)),
            pl.BlockSpec((D, D), lambda i: (0, 0)),
            pl.BlockSpec((D, D), lambda i: (0, 0)),
            pl.BlockSpec((1, D), lambda i: (0, 0)),
        ],
        out_specs=pl.BlockSpec((ROW_BLK, D), lambda i: (i, 0)),
        out_shape=jax.ShapeDtypeStruct((NN, D), f32),
    )

    edge_call = _make_edge_kernel(n_sup)

    num_layers = Wm.shape[0]
    epres = [epre_call(ef, Wm[l, 2 * D:], bm[l][None])
             for l in range(num_layers)]
    for l in range(num_layers):
        wsd = jnp.stack([Wm[l, :D], Wm[l, D:2 * D]])
        table = table_call(x, wsd)
        agg2 = edge_call(table, epres[l], gidx, didx, env, zeros)
        x = update_call(x, agg2[0], agg2[1], Wa[l, :D], Wa[l, D:],
                        ba[l][None])
    return x


# speedup vs baseline: 1.2719x; 1.2719x over previous
import functools

import jax
import jax.numpy as jnp
from jax import lax
from jax.experimental import pallas as pl
from jax.experimental.pallas import tpu as pltpu
from jax.experimental.pallas import tpu_sc as plsc

CUTOFF = 10.0
D = 128
NC, NS, L = 2, 16, 16
NW = NC * NS
CHUNK = 48
SUP = 2
NN = 10000
AGG_PAD = 10112
ROW_BLK = 400
EF_BLK = 6720


def _envelope_body(d_ref, c_ref):
    d = d_ref[...]
    c = 0.5 * (jnp.cos(d * (jnp.pi / CUTOFF)) + 1.0)
    c_ref[...] = c * (d < CUTOFF).astype(jnp.float32)


def _epre_body(ef_ref, w_ref, b_ref, o_ref):
    o_ref[...] = (
        jnp.dot(ef_ref[...], w_ref[...], preferred_element_type=jnp.float32)
        + b_ref[...]
    )


def _table_body(x_ref, w_ref, o_ref):
    o_ref[...] = jnp.dot(x_ref[...], w_ref[0],
                         preferred_element_type=jnp.float32)


def _update_body(x_ref, a0_ref, a1_ref, wx_ref, wg_ref, b_ref, o_ref):
    h = (
        jnp.dot(x_ref[...], wx_ref[...], preferred_element_type=jnp.float32)
        + jnp.dot(a0_ref[...] + a1_ref[...], wg_ref[...],
                  preferred_element_type=jnp.float32)
        + b_ref[...]
    )
    o_ref[...] = jnp.maximum(h, 0.0)


def _splat(v16, i):
    return lax.gather(
        v16, jnp.full((L, 1), i, jnp.int32),
        lax.GatherDimensionNumbers(offset_dims=(), collapsed_slice_dims=(0,),
                                   start_index_map=(0,)),
        slice_sizes=(1,), mode=lax.GatherScatterMode.PROMISE_IN_BOUNDS)


def _edge_body(t_hbm, epre_hbm, gidx_hbm, didx_hbm, c_hbm, zero_hbm,
               out_hbm, gbuf, ebuf, gidx, didx, cbuf, agg_sh,
               sem_g, sem_s, sem_i, *, n_sup):
    cid = lax.axis_index("c")
    sid = lax.axis_index("s")
    wid = cid * NS + sid
    nps = AGG_PAD // NS

    row0 = sid * nps
    pltpu.sync_copy(zero_hbm.at[pl.ds(row0, nps)], agg_sh.at[pl.ds(row0, nps)])
    plsc.subcore_barrier()

    chunk0 = wid * (n_sup * SUP)

    def fetch_idx(s, sp):
        r0 = chunk0 + s * SUP
        a = pltpu.async_copy(gidx_hbm.at[pl.ds(r0, SUP)], gidx.at[sp], sem_i)
        b = pltpu.async_copy(didx_hbm.at[pl.ds(r0, SUP)], didx.at[sp], sem_i)
        c = pltpu.async_copy(c_hbm.at[pl.ds(r0, SUP)], cbuf.at[sp], sem_i)
        return a, b, c

    def wait_idx(sp):
        pltpu.make_async_copy(gidx_hbm.at[pl.ds(0, SUP)], gidx.at[sp],
                              sem_i).wait()
        pltpu.make_async_copy(didx_hbm.at[pl.ds(0, SUP)], didx.at[sp],
                              sem_i).wait()
        pltpu.make_async_copy(c_hbm.at[pl.ds(0, SUP)], cbuf.at[sp],
                              sem_i).wait()

    def issue_ge(s, sp, j, p):
        r = chunk0 + s * SUP + j
        pltpu.async_copy(t_hbm.at[gidx.at[sp, j]], gbuf.at[p], sem_g)
        pltpu.async_copy(epre_hbm.at[pl.ds(r * CHUNK, CHUNK)], ebuf.at[p],
                         sem_g)

    def wait_ge(p):
        pltpu.make_async_copy(t_hbm.at[gidx.at[0, 0]], gbuf.at[p],
                              sem_g).wait()
        pltpu.make_async_copy(epre_hbm.at[pl.ds(0, CHUNK)], ebuf.at[p],
                              sem_g).wait()

    def issue_scatter(sp, j, p):
        pltpu.async_copy(ebuf.at[p], agg_sh.at[didx.at[sp, j]], sem_s,
                         add=True)

    def wait_scatter(sp, j, p):
        pltpu.make_async_copy(ebuf.at[p], agg_sh.at[didx.at[sp, j]],
                              sem_s).wait()

    def compute(sp, j, p):
        for g in range(CHUNK // L):
            cv16 = cbuf[sp, j, pl.ds(g * L, L)]

            def edge_fn(i, _):
                e = g * L + i
                cv = _splat(cv16, i)
                for f in range(D // L):
                    sl = pl.ds(f * L, L)
                    v = gbuf[p, 2 * e, sl] + gbuf[p, 2 * e + 1, sl] \
                        + ebuf[p, e, sl]
                    ebuf[p, e, sl] = jnp.maximum(v, 0.0) * cv
                return 0

            lax.fori_loop(0, L, edge_fn, 0)

    a, b, c = fetch_idx(0, 0)
    a.wait(); b.wait(); c.wait()
    issue_ge(0, 0, 0, 0)

    def sup_body(s, _):
        sp = lax.rem(s, 2)
        spn = 1 - sp
        for j in range(SUP):
            p = j % 2
            wait_ge(p)
            if j == 0:
                @pl.when(s > 0)
                def _():
                    wait_scatter(sp, SUP - 2, 1 - p)
                @pl.when(s + 1 < n_sup)
                def _():
                    fetch_idx(s + 1, spn)
            else:
                wait_scatter(sp, j - 1, 1 - p)
            if j < SUP - 1:
                issue_ge(s, sp, j + 1, 1 - p)
            elif j == SUP - 1:
                @pl.when(s + 1 < n_sup)
                def _():
                    wait_idx(spn)
                    issue_ge(s + 1, spn, 0, 1 - p)
            compute(sp, j, p)
            issue_scatter(sp, j, p)
        return 0

    lax.fori_loop(0, n_sup, sup_body, 0)
    wait_scatter(lax.rem(n_sup - 1, 2), SUP - 1, 1)
    plsc.subcore_barrier()
    pltpu.sync_copy(agg_sh.at[pl.ds(row0, nps)],
                    out_hbm.at[cid, pl.ds(row0, nps)])


def _make_edge_kernel(n_sup):
    mesh = plsc.VectorSubcoreMesh(core_axis_name="c", subcore_axis_name="s",
                                  num_cores=NC, num_subcores=NS)
    return pl.kernel(
        functools.partial(_edge_body, n_sup=n_sup),
        out_type=jax.ShapeDtypeStruct((NC, AGG_PAD, D), jnp.float32),
        mesh=mesh,
        scratch_types=[
            pltpu.VMEM((2, 2 * CHUNK, D), jnp.float32),
            pltpu.VMEM((2, CHUNK, D), jnp.float32),
            pltpu.VMEM((2, SUP, 2 * CHUNK), jnp.int32),
            pltpu.VMEM((2, SUP, CHUNK), jnp.int32),
            pltpu.VMEM((2, SUP, CHUNK), jnp.float32),
            pltpu.VMEM_SHARED((AGG_PAD, D), jnp.float32),
            pltpu.SemaphoreType.DMA,
            pltpu.SemaphoreType.DMA,
            pltpu.SemaphoreType.DMA,
        ],
    )


def kernel(node_feats, edge_feats, edge_index, dist, Wm, bm, Wa, ba):
    n, d_in = node_feats.shape
    e, d_edge = edge_feats.shape
    f32 = jnp.float32

    n_sup = -(-e // (NW * SUP * CHUNK))
    e_pad = NW * SUP * CHUNK * n_sup
    n_chunk_rows = e_pad // CHUNK

    x = node_feats
    ef = jnp.pad(edge_feats, ((0, e_pad - e), (0, 0)))
    src = jnp.pad(edge_index[0], (0, e_pad - e))
    dst = jnp.pad(edge_index[1], (0, e_pad - e))
    distp = jnp.pad(dist, (0, e_pad - e), constant_values=2.0 * CUTOFF)
    zeros = jnp.zeros((AGG_PAD, D), f32)

    gidx = jnp.stack([src, NN + dst], axis=-1).reshape(n_chunk_rows, 2 * CHUNK)
    didx = dst.reshape(n_chunk_rows, CHUNK)

    env = pl.pallas_call(
        _envelope_body,
        out_shape=jax.ShapeDtypeStruct((e_pad // D, D), f32),
    )(distp.reshape(e_pad // D, D))
    env = env.reshape(n_chunk_rows, CHUNK)

    n_row_blocks = NN // ROW_BLK
    table_call = pl.pallas_call(
        _table_body,
        grid=(2, n_row_blocks),
        in_specs=[
            pl.BlockSpec((ROW_BLK, D), lambda c, i: (i, 0)),
            pl.BlockSpec((1, D, D), lambda c, i: (c, 0, 0)),
        ],
        out_specs=pl.BlockSpec((ROW_BLK, D),
                               lambda c, i: (c * (NN // ROW_BLK) + i, 0)),
        out_shape=jax.ShapeDtypeStruct((2 * NN, D), f32),
    )

    epre_call = pl.pallas_call(
        _epre_body,
        grid=(e_pad // EF_BLK,),
        in_specs=[
            pl.BlockSpec((EF_BLK, d_edge), lambda i: (i, 0)),
            pl.BlockSpec((d_edge, D), lambda i: (0, 0)),
            pl.BlockSpec((1, D), lambda i: (0, 0)),
        ],
        out_specs=pl.BlockSpec((EF_BLK, D), lambda i: (i, 0)),
        out_shape=jax.ShapeDtypeStruct((e_pad, D), f32),
    )

    update_call = pl.pallas_call(
        _update_body,
        grid=(n_row_blocks,),
        in_specs=[
            pl.BlockSpec((ROW_BLK, D), lambda i: (i, 0)),
            pl.BlockSpec((ROW_BLK, D), lambda i: (i, 0)),
            pl.BlockSpec((ROW_BLK, D), lambda i: (i, 0)),
            pl.BlockSpec((D, D), lambda i: (0, 0)),
            pl.BlockSpec((D, D), lambda i: (0, 0)),
            pl.BlockSpec((1, D), lambda i: (0, 0)),
        ],
        out_specs=pl.BlockSpec((ROW_BLK, D), lambda i: (i, 0)),
        out_shape=jax.ShapeDtypeStruct((NN, D), f32),
    )

    edge_call = _make_edge_kernel(n_sup)

    num_layers = Wm.shape[0]
    epres = [epre_call(ef, Wm[l, 2 * D:], bm[l][None])
             for l in range(num_layers)]
    for l in range(num_layers):
        wsd = jnp.stack([Wm[l, :D], Wm[l, D:2 * D]])
        table = table_call(x, wsd)
        agg2 = edge_call(table, epres[l], gidx, didx, env, zeros)
        x = update_call(x, agg2[0], agg2[1], Wa[l, :D], Wa[l, D:],
                        ba[l][None])
    return x
